# Initial kernel scaffold; baseline (speedup 1.0000x reference)
import jax
import jax.numpy as jnp
from jax.experimental import pallas as pl


def kernel(x, edge_src, edge_dst, W1, a1_src, a1_dst, W2, a2_src, a2_dst):
    raise NotImplementedError("write your pallas kernel here")


# speedup vs baseline: 10.0185x; 10.0185x over previous
import functools

import jax
import jax.numpy as jnp
from jax import lax
from jax.experimental import pallas as pl
from jax.experimental.pallas import tpu as pltpu
from jax.experimental.pallas import tpu_sc as plsc

N_NODES = 10000
NPAD = 10240
CB = 32
CBD = 128
NTILES = 32
EBLK = CBD * NTILES
BR = 256
TC_GRID = NPAD // BR


def _proj_kernel(x_ref, w_ref, ts_ref, td_ref, xw_ref, tso_ref, tdo_ref):
    xw = jnp.dot(x_ref[...], w_ref[...], preferred_element_type=jnp.float32)
    xw_ref[...] = xw
    tso_ref[...] = jnp.dot(xw, ts_ref[...], preferred_element_type=jnp.float32)
    tdo_ref[...] = jnp.dot(xw, td_ref[...], preferred_element_type=jnp.float32)


def _project(x_p, Wr, TS, TD):
    f = x_p.shape[1]
    c = Wr.shape[1]
    return pl.pallas_call(
        _proj_kernel,
        grid=(TC_GRID,),
        in_specs=[
            pl.BlockSpec((BR, f), lambda i: (i, 0)),
            pl.BlockSpec((f, c), lambda i: (0, 0)),
            pl.BlockSpec((c, 128), lambda i: (0, 0)),
            pl.BlockSpec((c, 128), lambda i: (0, 0)),
        ],
        out_specs=[
            pl.BlockSpec((BR, c), lambda i: (i, 0)),
            pl.BlockSpec((BR, 128), lambda i: (i, 0)),
            pl.BlockSpec((BR, 128), lambda i: (i, 0)),
        ],
        out_shape=[
            jax.ShapeDtypeStruct((NPAD, c), jnp.float32),
            jax.ShapeDtypeStruct((NPAD, 128), jnp.float32),
            jax.ShapeDtypeStruct((NPAD, 128), jnp.float32),
        ],
    )(x_p, Wr, TS, TD)


def _combine_kernel(accp_ref, denp_ref, w2_ref, ts_ref, td_ref,
                    xw2_ref, tso_ref, tdo_ref):
    acc = accp_ref[0] + accp_ref[1]
    den = denp_ref[0] + denp_ref[1]
    cols = []
    for h in range(4):
        cols.append(acc[:, h * 32:(h + 1) * 32] / (den[:, h:h + 1] + 1e-9))
    hfeat = jnp.maximum(jnp.concatenate(cols, axis=1), 0.0)
    xw2 = jnp.dot(hfeat, w2_ref[...], preferred_element_type=jnp.float32)
    xw2_ref[...] = jnp.concatenate(
        [xw2, jnp.zeros((xw2.shape[0], 96), jnp.float32)], axis=1)
    tso_ref[...] = jnp.dot(xw2, ts_ref[...], preferred_element_type=jnp.float32)
    tdo_ref[...] = jnp.dot(xw2, td_ref[...], preferred_element_type=jnp.float32)


def _combine(accp, denp, W2r, TS2, TD2):
    return pl.pallas_call(
        _combine_kernel,
        grid=(TC_GRID,),
        in_specs=[
            pl.BlockSpec((2, BR, 128), lambda i: (0, i, 0)),
            pl.BlockSpec((2, BR, 128), lambda i: (0, i, 0)),
            pl.BlockSpec((128, 32), lambda i: (0, 0)),
            pl.BlockSpec((32, 128), lambda i: (0, 0)),
            pl.BlockSpec((32, 128), lambda i: (0, 0)),
        ],
        out_specs=[
            pl.BlockSpec((BR, 128), lambda i: (i, 0)),
            pl.BlockSpec((BR, 128), lambda i: (i, 0)),
            pl.BlockSpec((BR, 128), lambda i: (i, 0)),
        ],
        out_shape=[
            jax.ShapeDtypeStruct((NPAD, 128), jnp.float32),
            jax.ShapeDtypeStruct((NPAD, 128), jnp.float32),
            jax.ShapeDtypeStruct((NPAD, 128), jnp.float32),
        ],
    )(accp, denp, W2r, TS2, TD2)


def _final_kernel(accp_ref, denp_ref, out_ref):
    acc = accp_ref[0, :, :32] + accp_ref[1, :, :32]
    den = denp_ref[0, :, 0:1] + denp_ref[1, :, 0:1]
    out_ref[...] = acc / (den + 1e-9)


def _finalize(accp, denp):
    return pl.pallas_call(
        _final_kernel,
        grid=(TC_GRID,),
        in_specs=[
            pl.BlockSpec((2, BR, 128), lambda i: (0, i, 0)),
            pl.BlockSpec((2, BR, 128), lambda i: (0, i, 0)),
        ],
        out_specs=pl.BlockSpec((BR, 32), lambda i: (i, 0)),
        out_shape=jax.ShapeDtypeStruct((NPAD, 32), jnp.float32),
    )(accp, denp)


def _edge_phase(nheads, rowlen, nchunks):
    NHALF = 5120
    ACCR = 5248
    dlen = NHALF * nheads + 128
    mesh = plsc.VectorSubcoreMesh(core_axis_name="c", subcore_axis_name="s")

    @functools.partial(
        pl.kernel,
        out_type=(
            jax.ShapeDtypeStruct((2, NPAD, rowlen), jnp.float32),
            jax.ShapeDtypeStruct((2, NPAD, 128), jnp.float32),
        ),
        mesh=mesh,
        compiler_params=pltpu.CompilerParams(needs_layout_passes=False),
        scratch_types=(
            pltpu.VMEM((CB,), jnp.int32),
            pltpu.VMEM((CB,), jnp.int32),
            pltpu.VMEM((CB,), jnp.int32),
            pltpu.VMEM((CB, 128), jnp.float32),
            pltpu.VMEM((CB, 128), jnp.float32),
            pltpu.VMEM((CB, 128), jnp.float32),
            pltpu.VMEM((CB, rowlen), jnp.float32),
            pltpu.VMEM((CB, 128), jnp.float32),
            pltpu.VMEM_SHARED((ACCR, rowlen), jnp.float32),
            pltpu.VMEM_SHARED((ACCR, 128), jnp.float32),
            pltpu.SemaphoreType.DMA,
            pltpu.SemaphoreType.DMA,
            pltpu.SemaphoreType.DMA,
        ),
    )
    def edge_kernel(src_h, dst_h, dstr_h, ts_h, td_h, xw_h, outp_h, denp_h,
                    didx2, sidx2, didx32, gs, gd, rows, msg, wpad,
                    acc_sh, dacc_sh, sem0, sem1, sem2):
        c = lax.axis_index("c")
        s = lax.axis_index("s")
        g = c * 16 + s
        zero16 = jnp.zeros((16,), jnp.float32)
        iota16 = lax.iota(jnp.int32, 16)

        def ppass(p, _):
            def zloop(i, _):
                for j in range(rowlen // 16):
                    msg[i, pl.ds(j * 16, 16)] = zero16
                for j in range(8):
                    wpad[i, pl.ds(j * 16, 16)] = zero16
                return 0
            lax.fori_loop(0, CB, zloop, 0)

            def zcopy(k, _):
                r0 = s * 328 + k * CB
                pltpu.sync_copy(msg, acc_sh.at[pl.ds(r0, CB)])
                pltpu.sync_copy(wpad, dacc_sh.at[pl.ds(r0, CB)])
                return 0
            lax.fori_loop(0, 328 // CB, zcopy, 0)
            pltpu.sync_copy(msg.at[pl.ds(0, 8)],
                            acc_sh.at[pl.ds(s * 328 + 320, 8)])
            pltpu.sync_copy(wpad.at[pl.ds(0, 8)],
                            dacc_sh.at[pl.ds(s * 328 + 320, 8)])
            plsc.subcore_barrier()

            def chunk(k, _):
                rbase = (k * NTILES + g) * (CBD // CB)

                def sub(j, _):
                    pltpu.sync_copy(src_h.at[rbase + j], sidx2)
                    pltpu.sync_copy(dst_h.at[rbase + j], didx32)
                    pltpu.sync_copy(dstr_h.at[p, rbase + j], didx2)
                    cp0 = pltpu.async_copy(ts_h.at[sidx2], gs, sem0)
                    cp1 = pltpu.async_copy(td_h.at[didx32], gd, sem1)
                    cp2 = pltpu.async_copy(xw_h.at[sidx2], rows, sem2)
                    cp0.wait()
                    cp1.wait()
                    cp2.wait()

                    def edge(e, _):
                        ev = gs[e, pl.ds(0, 16)] + gd[e, pl.ds(0, 16)]
                        ev = jnp.where(ev > 0, ev, 0.2 * ev)
                        wv = jnp.exp(ev)
                        wv = jnp.where(iota16 < nheads, wv, 0.0)
                        wpad[e, pl.ds(0, 16)] = wv
                        for h in range(nheads):
                            ws = wv[h]
                            for jj in range(2):
                                sl = pl.ds(h * 32 + jj * 16, 16)
                                msg[e, sl] = rows[e, sl] * ws
                        return 0
                    lax.fori_loop(0, CB, edge, 0)

                    pltpu.sync_copy(msg, acc_sh.at[didx2], add=True)
                    pltpu.sync_copy(wpad, dacc_sh.at[didx2], add=True)
                    return 0
                lax.fori_loop(0, CBD // CB, sub, 0)
                return 0
            lax.fori_loop(0, nchunks, chunk, 0)
            plsc.subcore_barrier()

            def drain(k, _):
                r0 = s * 320 + k * 64
                pltpu.sync_copy(acc_sh.at[pl.ds(r0, 64)],
                                outp_h.at[c, pl.ds(p * NHALF + r0, 64)])
                pltpu.sync_copy(dacc_sh.at[pl.ds(r0, 64)],
                                denp_h.at[c, pl.ds(p * NHALF + r0, 64)])
                return 0
            lax.fori_loop(0, 5, drain, 0)
            plsc.subcore_barrier()
            return 0
        lax.fori_loop(0, 2, ppass, 0)

    return edge_kernel


def kernel(x, edge_src, edge_dst, W1, a1_src, a1_dst, W2, a2_src, a2_dst):
    n, f = x.shape
    heads, _, units = W1.shape
    e = edge_src.shape[0]

    x_p = jnp.zeros((NPAD, f), jnp.float32).at[:n].set(x)
    W1r = jnp.transpose(W1, (1, 0, 2)).reshape(f, heads * units)
    ridx = jnp.arange(heads * units)
    TS1 = jnp.zeros((heads * units, 128), jnp.float32).at[ridx, ridx // units].set(a1_src.reshape(-1))
    TD1 = jnp.zeros((heads * units, 128), jnp.float32).at[ridx, ridx // units].set(a1_dst.reshape(-1))
    W2r = W2.reshape(f, units)
    r2 = jnp.arange(units)
    TS2 = jnp.zeros((units, 128), jnp.float32).at[r2, 0].set(a2_src.reshape(-1))
    TD2 = jnp.zeros((units, 128), jnp.float32).at[r2, 0].set(a2_dst.reshape(-1))

    epad = ((e + EBLK - 1) // EBLK) * EBLK
    src_p = jnp.full((epad,), n, jnp.int32).at[:e].set(edge_src)
    dst_p = jnp.full((epad,), n, jnp.int32).at[:e].set(edge_dst)
    nchunks = epad // EBLK
    src3 = src_p.reshape(-1, CB)
    dst3 = dst_p.reshape(-1, CB)
    d0 = jnp.where(dst_p < 5120, dst_p, 5120)
    d1 = jnp.where(dst_p >= 5120, dst_p - 5120, 5120)
    dstr3 = jnp.stack([d0, d1]).reshape(2, -1, CB)

    xw1, ts1, td1 = _project(x_p, W1r, TS1, TD1)
    acc1, den1 = _edge_phase(4, 128, nchunks)(src3, dst3, dstr3, ts1, td1, xw1)


    xw2, ts2, td2 = _combine(acc1, den1, W2r, TS2, TD2)
    acc2, den2 = _edge_phase(1, 128, nchunks)(src3, dst3, dstr3, ts2, td2, xw2)


    out = _finalize(acc2, den2)
    return out[:n]
